# Initial kernel scaffold; baseline (speedup 1.0000x reference)
import jax
import jax.numpy as jnp
from jax.experimental import pallas as pl


def kernel(x, edge_index, adj_vals, W, b):
    raise NotImplementedError("write your pallas kernel here")


# speedup vs baseline: 3.3964x; 3.3964x over previous
import jax
import jax.numpy as jnp
from jax import lax
from jax.experimental import pallas as pl
from jax.experimental.pallas import tpu as pltpu
from jax.experimental.pallas import tpu_sc as plsc

N = 10000
E = 320000
D = 128

NUM_CORES = 2
NUM_TILES = 16
NUM_WORKERS = NUM_CORES * NUM_TILES
CHUNK = 128
CHUNKS_PER_WORKER = 80
EDGES_PER_WORKER = CHUNK * CHUNKS_PER_WORKER
E_PAD = EDGES_PER_WORKER * NUM_WORKERS
N_PAD = 10240
ROWS_PER_TILE = N_PAD // NUM_TILES
ZROWS = 128


def _spmm_kernel(x_hbm, src_hbm, dst_hbm, adj_hbm, out_hbm,
                 src_v, dst_v, adj_v, rows_v, acc_sh, sem):
    cid = lax.axis_index("c")
    sid = lax.axis_index("s")
    wid = cid * NUM_TILES + sid

    def zrow(r, _):
        z = jnp.zeros((16,), jnp.float32)
        for c in range(D // 16):
            rows_v[r, pl.ds(c * 16, 16)] = z
        return 0
    lax.fori_loop(0, ZROWS, zrow, 0, unroll=4)
    row0 = sid * ROWS_PER_TILE
    for p in range(ROWS_PER_TILE // ZROWS):
        pltpu.sync_copy(rows_v, acc_sh.at[pl.ds(row0 + p * ZROWS, ZROWS)])
    plsc.subcore_barrier()

    c0 = wid * CHUNKS_PER_WORKER
    pltpu.sync_copy(src_hbm.at[pl.ds(c0, CHUNKS_PER_WORKER)], src_v)
    pltpu.sync_copy(dst_hbm.at[pl.ds(c0, CHUNKS_PER_WORKER)], dst_v)
    pltpu.sync_copy(adj_hbm.at[pl.ds(c0, CHUNKS_PER_WORKER)], adj_v)

    def chunk_body(j, _):
        pltpu.async_copy(x_hbm.at[src_v.at[j]], rows_v, sem).wait()
        for g in range(CHUNK // 16):
            a16 = adj_v[j, pl.ds(g * 16, 16)]
            for l in range(16):
                av = jnp.full((16,), a16[l], jnp.float32)
                r = g * 16 + l
                for c in range(D // 16):
                    sl = pl.ds(c * 16, 16)
                    rows_v[r, sl] = rows_v[r, sl] * av
        pltpu.sync_copy(rows_v, acc_sh.at[dst_v.at[j]], add=True)
        return 0

    lax.fori_loop(0, CHUNKS_PER_WORKER, chunk_body, 0)
    plsc.subcore_barrier()

    pltpu.sync_copy(acc_sh.at[pl.ds(row0, ROWS_PER_TILE)],
                    out_hbm.at[cid].at[pl.ds(row0, ROWS_PER_TILE)])


@jax.jit
def _spmm(x, src2, dst2, adj2):
    mesh = plsc.VectorSubcoreMesh(core_axis_name="c", subcore_axis_name="s")
    return pl.kernel(
        _spmm_kernel,
        mesh=mesh,
        out_type=jax.ShapeDtypeStruct((NUM_CORES, N_PAD, D), jnp.float32),
        scratch_types=[
            pltpu.VMEM((CHUNKS_PER_WORKER, CHUNK), jnp.int32),
            pltpu.VMEM((CHUNKS_PER_WORKER, CHUNK), jnp.int32),
            pltpu.VMEM((CHUNKS_PER_WORKER, CHUNK), jnp.float32),
            pltpu.VMEM((CHUNK, D), jnp.float32),
            pltpu.MemorySpace.VMEM_SHARED((N_PAD, D), jnp.float32),
            pltpu.SemaphoreType.DMA,
        ],
    )(x, src2, dst2, adj2)


def _finish_body(p0_ref, p1_ref, w_ref, b_ref, o_ref):
    s = p0_ref[...] + p1_ref[...]
    y = jnp.dot(s, w_ref[...], preferred_element_type=jnp.float32)
    o_ref[...] = jnp.maximum(y + b_ref[...], 0.0)


@jax.jit
def _finish(p0, p1, W, b2):
    blk = 400
    grid = (N // blk,)
    return pl.pallas_call(
        _finish_body,
        grid=grid,
        in_specs=[
            pl.BlockSpec((blk, D), lambda i: (i, 0)),
            pl.BlockSpec((blk, D), lambda i: (i, 0)),
            pl.BlockSpec((D, D), lambda i: (0, 0)),
            pl.BlockSpec((1, D), lambda i: (0, 0)),
        ],
        out_specs=pl.BlockSpec((blk, D), lambda i: (i, 0)),
        out_shape=jax.ShapeDtypeStruct((N, D), jnp.float32),
    )(p0, p1, W, b2)


def kernel(x, edge_index, adj_vals, W, b):
    pad = E_PAD - E
    src2 = jnp.concatenate(
        [edge_index[0], jnp.zeros((pad,), jnp.int32)]).reshape(-1, CHUNK)
    dst2 = jnp.concatenate(
        [edge_index[1], jnp.zeros((pad,), jnp.int32)]).reshape(-1, CHUNK)
    adj2 = jnp.concatenate(
        [adj_vals, jnp.zeros((pad,), jnp.float32)]).reshape(-1, CHUNK)
    partials = _spmm(x, src2, dst2, adj2)
    return _finish(partials[0, :N], partials[1, :N], W, b.reshape(1, D))


# speedup vs baseline: 4.1303x; 1.2161x over previous
import jax
import jax.numpy as jnp
from jax import lax
from jax.experimental import pallas as pl
from jax.experimental.pallas import tpu as pltpu
from jax.experimental.pallas import tpu_sc as plsc

N = 10000
E = 320000
D = 128

NUM_CORES = 2
NUM_TILES = 16
NUM_WORKERS = NUM_CORES * NUM_TILES
UNIT = 64
PUNITS = 40
PHASES = 4
UNITS_PER_WORKER = PUNITS * PHASES
EDGES_PER_WORKER = UNIT * UNITS_PER_WORKER
E_PAD = EDGES_PER_WORKER * NUM_WORKERS
N_PAD = 10240
ROWS_PER_TILE = N_PAD // NUM_TILES
NBUF = 4


def _scale_unit(buf, adj_v, j):
    def group(g, _):
        a16 = adj_v[j, pl.ds(g * 16, 16)]
        for l in range(16):
            av = jnp.full((16,), a16[l], jnp.float32)
            r = g * 16 + l
            for c in range(D // 16):
                sl = pl.ds(c * 16, 16)
                buf[r, sl] = buf[r, sl] * av
        return 0
    lax.fori_loop(0, UNIT // 16, group, 0)


def _spmm_kernel(x_hbm, src_hbm, dst_hbm, adj_hbm, out_hbm,
                 src_v, dst_v, adj_v, b0, b1, b2, b3,
                 g0, g1, g2, g3, s0, s1, s2, s3, acc_sh):
    bufs = (b0, b1, b2, b3)
    gsems = (g0, g1, g2, g3)
    ssems = (s0, s1, s2, s3)
    cid = lax.axis_index("c")
    sid = lax.axis_index("s")
    wid = cid * NUM_TILES + sid

    def zrow(r, _):
        z = jnp.zeros((16,), jnp.float32)
        for c in range(D // 16):
            b0[r, pl.ds(c * 16, 16)] = z
        return 0
    lax.fori_loop(0, UNIT, zrow, 0, unroll=4)
    row0 = sid * ROWS_PER_TILE
    for p in range(ROWS_PER_TILE // UNIT):
        pltpu.sync_copy(b0, acc_sh.at[pl.ds(row0 + p * UNIT, UNIT)])
    plsc.subcore_barrier()

    def phase_body(phase, _):
        u0 = wid * UNITS_PER_WORKER + phase * PUNITS
        pltpu.sync_copy(src_hbm.at[pl.ds(u0, PUNITS)], src_v)
        pltpu.sync_copy(dst_hbm.at[pl.ds(u0, PUNITS)], dst_v)
        pltpu.sync_copy(adj_hbm.at[pl.ds(u0, PUNITS)], adj_v)

        pltpu.async_copy(x_hbm.at[src_v.at[0]], b0, g0)
        pltpu.async_copy(x_hbm.at[src_v.at[1]], b1, g1)

        def unit_body(jj, _):
            for k in range(NBUF):
                j = NBUF * jj + k
                kn = (k + 2) % NBUF

                @pl.when(j + 2 < PUNITS)
                def _():
                    @pl.when(j >= 2)
                    def _():
                        pltpu.make_async_copy(
                            bufs[kn], acc_sh.at[dst_v.at[j - 2]],
                            ssems[kn]).wait()
                    pltpu.async_copy(x_hbm.at[src_v.at[j + 2]],
                                     bufs[kn], gsems[kn])

                pltpu.make_async_copy(x_hbm.at[src_v.at[j]],
                                      bufs[k], gsems[k]).wait()
                _scale_unit(bufs[k], adj_v, j)
                pltpu.async_copy(bufs[k], acc_sh.at[dst_v.at[j]],
                                 ssems[k], add=True)
            return 0

        lax.fori_loop(0, PUNITS // NBUF, unit_body, 0)
        for k in range(NBUF):
            pltpu.make_async_copy(bufs[k], acc_sh.at[dst_v.at[0]],
                                  ssems[k]).wait()
        return 0

    lax.fori_loop(0, PHASES, phase_body, 0)
    plsc.subcore_barrier()

    pltpu.sync_copy(acc_sh.at[pl.ds(row0, ROWS_PER_TILE)],
                    out_hbm.at[cid].at[pl.ds(row0, ROWS_PER_TILE)])


@jax.jit
def _spmm(x, src2, dst2, adj2):
    mesh = plsc.VectorSubcoreMesh(core_axis_name="c", subcore_axis_name="s")
    return pl.kernel(
        _spmm_kernel,
        mesh=mesh,
        out_type=jax.ShapeDtypeStruct((NUM_CORES, N_PAD, D), jnp.float32),
        scratch_types=[
            pltpu.VMEM((PUNITS, UNIT), jnp.int32),
            pltpu.VMEM((PUNITS, UNIT), jnp.int32),
            pltpu.VMEM((PUNITS, UNIT), jnp.float32),
            pltpu.VMEM((UNIT, D), jnp.float32),
            pltpu.VMEM((UNIT, D), jnp.float32),
            pltpu.VMEM((UNIT, D), jnp.float32),
            pltpu.VMEM((UNIT, D), jnp.float32),
            pltpu.SemaphoreType.DMA,
            pltpu.SemaphoreType.DMA,
            pltpu.SemaphoreType.DMA,
            pltpu.SemaphoreType.DMA,
            pltpu.SemaphoreType.DMA,
            pltpu.SemaphoreType.DMA,
            pltpu.SemaphoreType.DMA,
            pltpu.SemaphoreType.DMA,
            pltpu.MemorySpace.VMEM_SHARED((N_PAD, D), jnp.float32),
        ],
    )(x, src2, dst2, adj2)


def _finish_body(p0_ref, p1_ref, w_ref, b_ref, o_ref):
    s = p0_ref[...] + p1_ref[...]
    y = jnp.dot(s, w_ref[...], preferred_element_type=jnp.float32)
    o_ref[...] = jnp.maximum(y + b_ref[...], 0.0)


@jax.jit
def _finish(p0, p1, W, b2):
    blk = 400
    grid = (N // blk,)
    return pl.pallas_call(
        _finish_body,
        grid=grid,
        in_specs=[
            pl.BlockSpec((blk, D), lambda i: (i, 0)),
            pl.BlockSpec((blk, D), lambda i: (i, 0)),
            pl.BlockSpec((D, D), lambda i: (0, 0)),
            pl.BlockSpec((1, D), lambda i: (0, 0)),
        ],
        out_specs=pl.BlockSpec((blk, D), lambda i: (i, 0)),
        out_shape=jax.ShapeDtypeStruct((N, D), jnp.float32),
    )(p0, p1, W, b2)


def kernel(x, edge_index, adj_vals, W, b):
    pad = E_PAD - E
    src2 = jnp.concatenate(
        [edge_index[0], jnp.zeros((pad,), jnp.int32)]).reshape(-1, UNIT)
    dst2 = jnp.concatenate(
        [edge_index[1], jnp.zeros((pad,), jnp.int32)]).reshape(-1, UNIT)
    adj2 = jnp.concatenate(
        [adj_vals, jnp.zeros((pad,), jnp.float32)]).reshape(-1, UNIT)
    partials = _spmm(x, src2, dst2, adj2)
    return _finish(partials[0, :N], partials[1, :N], W, b.reshape(1, D))


# speedup vs baseline: 4.2794x; 1.0361x over previous
import jax
import jax.numpy as jnp
from jax import lax
from jax.experimental import pallas as pl
from jax.experimental.pallas import tpu as pltpu
from jax.experimental.pallas import tpu_sc as plsc

N = 10000
E = 320000
D = 128

NUM_CORES = 2
NUM_TILES = 16
UNIT = 64
PUNITS = 40
UNITS_CORE0 = 240
UNITS_CORE1 = 80
PHASES0 = UNITS_CORE0 // PUNITS
PHASES1 = UNITS_CORE1 // PUNITS
UNITS_TOTAL = UNITS_CORE0 + UNITS_CORE1
E_PAD = UNIT * UNITS_TOTAL * NUM_TILES
N_PAD = 10240
ROWS_PER_TILE = N_PAD // NUM_TILES
NBUF = 4


def _scale_unit(buf, adj_v, j):
    def group(g, _):
        a16 = adj_v[j, pl.ds(g * 16, 16)]
        for l in range(16):
            av = jnp.full((16,), a16[l], jnp.float32)
            r = g * 16 + l
            for c in range(D // 16):
                sl = pl.ds(c * 16, 16)
                buf[r, sl] = buf[r, sl] * av
        return 0
    lax.fori_loop(0, UNIT // 16, group, 0)


def _spmm_kernel(x_hbm, src_hbm, dst_hbm, adj_hbm, out_hbm,
                 src_v, dst_v, adj_v, b0, b1, b2, b3,
                 g0, g1, g2, g3, s0, s1, s2, s3, acc_sh):
    bufs = (b0, b1, b2, b3)
    gsems = (g0, g1, g2, g3)
    ssems = (s0, s1, s2, s3)
    cid = lax.axis_index("c")
    sid = lax.axis_index("s")

    def zrow(r, _):
        z = jnp.zeros((16,), jnp.float32)
        for c in range(D // 16):
            b0[r, pl.ds(c * 16, 16)] = z
        return 0
    lax.fori_loop(0, UNIT, zrow, 0, unroll=4)
    row0 = sid * ROWS_PER_TILE
    for p in range(ROWS_PER_TILE // UNIT):
        pltpu.sync_copy(b0, acc_sh.at[pl.ds(row0 + p * UNIT, UNIT)])
    plsc.subcore_barrier()

    ubase = jnp.where(cid == 0, sid * UNITS_CORE0,
                      NUM_TILES * UNITS_CORE0 + sid * UNITS_CORE1)
    nphases = jnp.where(cid == 0, PHASES0, PHASES1)

    def phase_body(phase, _):
        u0 = ubase + phase * PUNITS
        pltpu.sync_copy(src_hbm.at[pl.ds(u0, PUNITS)], src_v)
        pltpu.sync_copy(dst_hbm.at[pl.ds(u0, PUNITS)], dst_v)
        pltpu.sync_copy(adj_hbm.at[pl.ds(u0, PUNITS)], adj_v)

        pltpu.async_copy(x_hbm.at[src_v.at[0]], b0, g0)
        pltpu.async_copy(x_hbm.at[src_v.at[1]], b1, g1)

        def unit_body(jj, _):
            for k in range(NBUF):
                j = NBUF * jj + k
                kn = (k + 2) % NBUF

                @pl.when(j + 2 < PUNITS)
                def _():
                    @pl.when(j >= 2)
                    def _():
                        pltpu.make_async_copy(
                            bufs[kn], acc_sh.at[dst_v.at[j - 2]],
                            ssems[kn]).wait()
                    pltpu.async_copy(x_hbm.at[src_v.at[j + 2]],
                                     bufs[kn], gsems[kn])

                pltpu.make_async_copy(x_hbm.at[src_v.at[j]],
                                      bufs[k], gsems[k]).wait()
                _scale_unit(bufs[k], adj_v, j)
                pltpu.async_copy(bufs[k], acc_sh.at[dst_v.at[j]],
                                 ssems[k], add=True)
            return 0

        lax.fori_loop(0, PUNITS // NBUF, unit_body, 0)
        for k in range(NBUF):
            pltpu.make_async_copy(bufs[k], acc_sh.at[dst_v.at[0]],
                                  ssems[k]).wait()
        return 0

    lax.fori_loop(0, nphases, phase_body, 0)
    plsc.subcore_barrier()

    pltpu.sync_copy(acc_sh.at[pl.ds(row0, ROWS_PER_TILE)],
                    out_hbm.at[cid].at[pl.ds(row0, ROWS_PER_TILE)])


@jax.jit
def _spmm(x, src2, dst2, adj2):
    mesh = plsc.VectorSubcoreMesh(core_axis_name="c", subcore_axis_name="s")
    return pl.kernel(
        _spmm_kernel,
        mesh=mesh,
        out_type=jax.ShapeDtypeStruct((NUM_CORES, N_PAD, D), jnp.float32),
        scratch_types=[
            pltpu.VMEM((PUNITS, UNIT), jnp.int32),
            pltpu.VMEM((PUNITS, UNIT), jnp.int32),
            pltpu.VMEM((PUNITS, UNIT), jnp.float32),
            pltpu.VMEM((UNIT, D), jnp.float32),
            pltpu.VMEM((UNIT, D), jnp.float32),
            pltpu.VMEM((UNIT, D), jnp.float32),
            pltpu.VMEM((UNIT, D), jnp.float32),
            pltpu.SemaphoreType.DMA,
            pltpu.SemaphoreType.DMA,
            pltpu.SemaphoreType.DMA,
            pltpu.SemaphoreType.DMA,
            pltpu.SemaphoreType.DMA,
            pltpu.SemaphoreType.DMA,
            pltpu.SemaphoreType.DMA,
            pltpu.SemaphoreType.DMA,
            pltpu.MemorySpace.VMEM_SHARED((N_PAD, D), jnp.float32),
        ],
    )(x, src2, dst2, adj2)


def _finish_body(p0_ref, p1_ref, w_ref, b_ref, o_ref):
    s = p0_ref[...] + p1_ref[...]
    y = jnp.dot(s, w_ref[...], preferred_element_type=jnp.float32)
    o_ref[...] = jnp.maximum(y + b_ref[...], 0.0)


@jax.jit
def _finish(p0, p1, W, b2):
    blk = 400
    grid = (N // blk,)
    return pl.pallas_call(
        _finish_body,
        grid=grid,
        in_specs=[
            pl.BlockSpec((blk, D), lambda i: (i, 0)),
            pl.BlockSpec((blk, D), lambda i: (i, 0)),
            pl.BlockSpec((D, D), lambda i: (0, 0)),
            pl.BlockSpec((1, D), lambda i: (0, 0)),
        ],
        out_specs=pl.BlockSpec((blk, D), lambda i: (i, 0)),
        out_shape=jax.ShapeDtypeStruct((N, D), jnp.float32),
    )(p0, p1, W, b2)


def kernel(x, edge_index, adj_vals, W, b):
    pad = E_PAD - E
    src2 = jnp.concatenate(
        [edge_index[0], jnp.zeros((pad,), jnp.int32)]).reshape(-1, UNIT)
    dst2 = jnp.concatenate(
        [edge_index[1], jnp.zeros((pad,), jnp.int32)]).reshape(-1, UNIT)
    adj2 = jnp.concatenate(
        [adj_vals, jnp.zeros((pad,), jnp.float32)]).reshape(-1, UNIT)
    partials = _spmm(x, src2, dst2, adj2)
    return _finish(partials[0, :N], partials[1, :N], W, b.reshape(1, D))
